# Initial kernel scaffold; baseline (speedup 1.0000x reference)
import jax
import jax.numpy as jnp
from jax.experimental import pallas as pl


def kernel(logits, labels):
    raise NotImplementedError("write your pallas kernel here")


# speedup vs baseline: 1.2518x; 1.2518x over previous
import jax
import jax.numpy as jnp
from jax.experimental import pallas as pl

_S = 64.0
_M = 0.4

_COLS_BLK = 2048


def _cosface_tile(labels_ref, x_ref, o_ref):
    j = pl.program_id(0)
    base = j * _COLS_BLK
    x = x_ref[...]
    labels = labels_ref[...]
    col = jax.lax.broadcasted_iota(jnp.int32, x.shape, 1) + base
    mask = (col == labels) & (labels >= 0)
    o_ref[...] = jnp.where(mask, x * _S - (_M * _S), x * _S)


def kernel(logits, labels):
    b, c = logits.shape
    labels2 = labels.astype(jnp.int32).reshape(b, 1)
    grid = (pl.cdiv(c, _COLS_BLK),)
    return pl.pallas_call(
        _cosface_tile,
        grid=grid,
        in_specs=[
            pl.BlockSpec((b, 1), lambda j: (0, 0)),
            pl.BlockSpec((b, _COLS_BLK), lambda j: (0, j)),
        ],
        out_specs=pl.BlockSpec((b, _COLS_BLK), lambda j: (0, j)),
        out_shape=jax.ShapeDtypeStruct((b, c), logits.dtype),
    )(labels2, logits)
